# baseline (speedup 1.0000x reference)
import functools

import jax
import jax.numpy as jnp
from jax import lax
from jax.experimental import pallas as pl
from jax.experimental.pallas import tpu as pltpu
from jax.experimental.pallas import tpu_sc as plsc

HIDDEN = 4096
N_EXPERTS = 64
TOP_K = 8
LANES = 16
NUM_WORKERS = 32

_TC_BLOCK_T = 1024


def _logits_body(x_ref, w_ref, out_ref):
    out_ref[...] = lax.dot_general(
        w_ref[...], x_ref[...],
        dimension_numbers=(((1,), (1,)), ((), ())),
        preferred_element_type=jnp.float32)


def _router_logits_t(flat, weight, start, size):
    blk_off = start // _TC_BLOCK_T
    return pl.pallas_call(
        _logits_body,
        grid=(size // _TC_BLOCK_T,),
        in_specs=[
            pl.BlockSpec((_TC_BLOCK_T, HIDDEN), lambda i: (i + blk_off, 0)),
            pl.BlockSpec((N_EXPERTS, HIDDEN), lambda i: (0, 0)),
        ],
        out_specs=pl.BlockSpec((N_EXPERTS, _TC_BLOCK_T), lambda i: (0, i)),
        out_shape=jax.ShapeDtypeStruct((N_EXPERTS, size), jnp.float32),
    )(flat, weight)


def _make_topk_sc(t):
    per_w = t // NUM_WORKERS
    n_groups = per_w // LANES

    mesh = plsc.VectorSubcoreMesh(core_axis_name="c", subcore_axis_name="s")

    @functools.partial(
        pl.kernel,
        out_type=(
            jax.ShapeDtypeStruct((TOP_K, t), jnp.int32),
            jax.ShapeDtypeStruct((TOP_K, t), jnp.float32),
        ),
        mesh=mesh,
        scratch_types=[
            pltpu.VMEM((N_EXPERTS, per_w), jnp.float32),
            pltpu.VMEM((TOP_K, per_w), jnp.int32),
            pltpu.VMEM((TOP_K, per_w), jnp.float32),
        ],
    )
    def topk_kernel(logits_hbm, idx_hbm, w_hbm, lv, iv, wv):
        wid = lax.axis_index("s") * 2 + lax.axis_index("c")
        base = wid * per_w
        pltpu.sync_copy(logits_hbm.at[:, pl.ds(base, per_w)], lv)

        def group_body(g, carry):
            goff = g * LANES
            tv = [jnp.full((LANES,), -jnp.inf, jnp.float32)
                  for _ in range(TOP_K)]
            ti = [jnp.zeros((LANES,), jnp.int32) for _ in range(TOP_K)]
            for e in range(N_EXPERTS - 1, -1, -1):
                v = lv[e, pl.ds(goff, LANES)]
                idx = jnp.full((LANES,), e, jnp.int32)
                for j in range(TOP_K):
                    m = v >= tv[j]
                    hi = jnp.maximum(tv[j], v)
                    lo = jnp.minimum(tv[j], v)
                    ni = jnp.where(m, idx, ti[j])
                    idx = jnp.where(m, ti[j], idx)
                    tv[j] = hi
                    ti[j] = ni
                    v = lo
            mx = tv[0]
            es = [jnp.exp(tv[j] - mx) for j in range(TOP_K)]
            s = es[0]
            for j in range(1, TOP_K):
                s = s + es[j]
            r = 1.0 / s
            for j in range(TOP_K):
                iv[j, pl.ds(goff, LANES)] = ti[j]
                wv[j, pl.ds(goff, LANES)] = es[j] * r
            return carry

        lax.fori_loop(0, n_groups, group_body, 0)
        pltpu.sync_copy(iv, idx_hbm.at[:, pl.ds(base, per_w)])
        pltpu.sync_copy(wv, w_hbm.at[:, pl.ds(base, per_w)])

    return topk_kernel


_STRIPE_SIZES = (28672, 4096)


def kernel(hidden_states, weight):
    bsz, seq_len, h = hidden_states.shape
    t = bsz * seq_len
    flat = hidden_states.reshape(t, h)
    topk_cache = {}
    idx_parts = []
    w_parts = []
    start = 0
    for size in _STRIPE_SIZES:
        if size not in topk_cache:
            topk_cache[size] = _make_topk_sc(size)
        logits_t = _router_logits_t(flat, weight, start, size)
        idx_t, w_t = topk_cache[size](logits_t)
        idx_parts.append(idx_t)
        w_parts.append(w_t)
        start += size
    topk_idx_t = jnp.concatenate(idx_parts, axis=1)
    topk_weight_t = jnp.concatenate(w_parts, axis=1)
    aux_loss = jnp.float32(0.0)
    return (topk_idx_t.T, topk_weight_t.T, aux_loss)


# speedup vs baseline: 1.0249x; 1.0249x over previous
import functools

import jax
import jax.numpy as jnp
from jax import lax
from jax.experimental import pallas as pl
from jax.experimental.pallas import tpu as pltpu
from jax.experimental.pallas import tpu_sc as plsc

HIDDEN = 4096
N_EXPERTS = 64
TOP_K = 8
LANES = 16
NUM_WORKERS = 32

_TC_BLOCK_T = 1024


def _logits_body(x_ref, w_ref, out_ref):
    out_ref[...] = lax.dot_general(
        w_ref[...], x_ref[...],
        dimension_numbers=(((1,), (1,)), ((), ())),
        preferred_element_type=jnp.float32)


def _router_logits_t(flat, weight, start, size):
    blk_off = start // _TC_BLOCK_T
    return pl.pallas_call(
        _logits_body,
        grid=(size // _TC_BLOCK_T,),
        in_specs=[
            pl.BlockSpec((_TC_BLOCK_T, HIDDEN), lambda i: (i + blk_off, 0)),
            pl.BlockSpec((N_EXPERTS, HIDDEN), lambda i: (0, 0)),
        ],
        out_specs=pl.BlockSpec((N_EXPERTS, _TC_BLOCK_T), lambda i: (0, i)),
        out_shape=jax.ShapeDtypeStruct((N_EXPERTS, size), jnp.float32),
    )(flat, weight)


def _make_topk_sc(t):
    per_w = t // NUM_WORKERS
    n_groups = per_w // LANES

    mesh = plsc.VectorSubcoreMesh(core_axis_name="c", subcore_axis_name="s")

    @functools.partial(
        pl.kernel,
        out_type=(
            jax.ShapeDtypeStruct((TOP_K, t), jnp.int32),
            jax.ShapeDtypeStruct((TOP_K, t), jnp.float32),
        ),
        mesh=mesh,
        scratch_types=[
            pltpu.VMEM((N_EXPERTS, per_w), jnp.float32),
            pltpu.VMEM((TOP_K, per_w), jnp.int32),
            pltpu.VMEM((TOP_K, per_w), jnp.float32),
        ],
    )
    def topk_kernel(logits_hbm, idx_hbm, w_hbm, lv, iv, wv):
        wid = lax.axis_index("s") * 2 + lax.axis_index("c")
        base = wid * per_w
        pltpu.sync_copy(logits_hbm.at[:, pl.ds(base, per_w)], lv)

        def group_body(g, carry):
            goff = g * LANES
            tv = [jnp.full((LANES,), -jnp.inf, jnp.float32)
                  for _ in range(TOP_K)]
            ti = [jnp.zeros((LANES,), jnp.int32) for _ in range(TOP_K)]
            for e in range(N_EXPERTS - 1, -1, -1):
                v = lv[e, pl.ds(goff, LANES)]
                idx = jnp.full((LANES,), e, jnp.int32)
                for j in range(TOP_K):
                    m = v >= tv[j]
                    hi = jnp.maximum(tv[j], v)
                    lo = jnp.minimum(tv[j], v)
                    ni = jnp.where(m, idx, ti[j])
                    idx = jnp.where(m, ti[j], idx)
                    tv[j] = hi
                    ti[j] = ni
                    v = lo
            mx = tv[0]
            es = [jnp.exp(tv[j] - mx) for j in range(TOP_K)]
            s = es[0]
            for j in range(1, TOP_K):
                s = s + es[j]
            r = 1.0 / s
            for j in range(TOP_K):
                iv[j, pl.ds(goff, LANES)] = ti[j]
                wv[j, pl.ds(goff, LANES)] = es[j] * r
            return carry

        lax.fori_loop(0, n_groups, group_body, 0)
        pltpu.sync_copy(iv, idx_hbm.at[:, pl.ds(base, per_w)])
        pltpu.sync_copy(wv, w_hbm.at[:, pl.ds(base, per_w)])

    return topk_kernel


_STRIPE_SIZES = (20480, 8192, 4096)


def kernel(hidden_states, weight):
    bsz, seq_len, h = hidden_states.shape
    t = bsz * seq_len
    flat = hidden_states.reshape(t, h)
    topk_cache = {}
    idx_parts = []
    w_parts = []
    start = 0
    for size in _STRIPE_SIZES:
        if size not in topk_cache:
            topk_cache[size] = _make_topk_sc(size)
        logits_t = _router_logits_t(flat, weight, start, size)
        idx_t, w_t = topk_cache[size](logits_t)
        idx_parts.append(idx_t.T)
        w_parts.append(w_t.T)
        start += size
    topk_idx = jnp.concatenate(idx_parts, axis=0)
    topk_weight = jnp.concatenate(w_parts, axis=0)
    aux_loss = jnp.float32(0.0)
    return (topk_idx, topk_weight, aux_loss)


# speedup vs baseline: 1.0269x; 1.0019x over previous
import functools

import jax
import jax.numpy as jnp
from jax import lax
from jax.experimental import pallas as pl
from jax.experimental.pallas import tpu as pltpu
from jax.experimental.pallas import tpu_sc as plsc

HIDDEN = 4096
N_EXPERTS = 64
TOP_K = 8
LANES = 16
NUM_WORKERS = 32

_TC_BLOCK_T = 1024


def _logits_body(x_ref, w_ref, out_ref):
    out_ref[...] = lax.dot_general(
        w_ref[...], x_ref[...],
        dimension_numbers=(((1,), (1,)), ((), ())),
        preferred_element_type=jnp.float32)


def _router_logits_t(flat, weight, start, size):
    blk_off = start // _TC_BLOCK_T
    return pl.pallas_call(
        _logits_body,
        grid=(size // _TC_BLOCK_T,),
        in_specs=[
            pl.BlockSpec((_TC_BLOCK_T, HIDDEN), lambda i: (i + blk_off, 0)),
            pl.BlockSpec((N_EXPERTS, HIDDEN), lambda i: (0, 0)),
        ],
        out_specs=pl.BlockSpec((N_EXPERTS, _TC_BLOCK_T), lambda i: (0, i)),
        out_shape=jax.ShapeDtypeStruct((N_EXPERTS, size), jnp.float32),
    )(flat, weight)


def _make_topk_sc(t):
    per_w = t // NUM_WORKERS
    n_groups = per_w // LANES

    mesh = plsc.VectorSubcoreMesh(core_axis_name="c", subcore_axis_name="s")

    @functools.partial(
        pl.kernel,
        out_type=(
            jax.ShapeDtypeStruct((TOP_K, t), jnp.int32),
            jax.ShapeDtypeStruct((TOP_K, t), jnp.float32),
        ),
        mesh=mesh,
        scratch_types=[
            pltpu.VMEM((N_EXPERTS, per_w), jnp.float32),
            pltpu.VMEM((TOP_K, per_w), jnp.int32),
            pltpu.VMEM((TOP_K, per_w), jnp.float32),
        ],
    )
    def topk_kernel(logits_hbm, idx_hbm, w_hbm, lv, iv, wv):
        wid = lax.axis_index("s") * 2 + lax.axis_index("c")
        base = wid * per_w
        pltpu.sync_copy(logits_hbm.at[:, pl.ds(base, per_w)], lv)

        def group_body(g, carry):
            goff = g * LANES

            def chunk_body(c, state):
                tv = list(state[:TOP_K])
                ti = list(state[TOP_K:])
                e_hi = (7 - c) * 8
                for k in range(7, -1, -1):
                    e = e_hi + k
                    v = lv[e, pl.ds(goff, LANES)]
                    idx = jnp.broadcast_to(e, (LANES,)).astype(jnp.int32)
                    for j in range(TOP_K):
                        m = v >= tv[j]
                        hi = jnp.maximum(tv[j], v)
                        lo = jnp.minimum(tv[j], v)
                        ni = jnp.where(m, idx, ti[j])
                        idx = jnp.where(m, ti[j], idx)
                        tv[j] = hi
                        ti[j] = ni
                        v = lo
                return tuple(tv) + tuple(ti)

            init = tuple(jnp.full((LANES,), -jnp.inf, jnp.float32)
                         for _ in range(TOP_K)) + \
                tuple(jnp.zeros((LANES,), jnp.int32) for _ in range(TOP_K))
            state = lax.fori_loop(0, 8, chunk_body, init)
            tv = list(state[:TOP_K])
            ti = list(state[TOP_K:])
            mx = tv[0]
            es = [jnp.exp(tv[j] - mx) for j in range(TOP_K)]
            s = es[0]
            for j in range(1, TOP_K):
                s = s + es[j]
            r = 1.0 / s
            for j in range(TOP_K):
                iv[j, pl.ds(goff, LANES)] = ti[j]
                wv[j, pl.ds(goff, LANES)] = es[j] * r
            return carry

        lax.fori_loop(0, n_groups, group_body, 0)
        pltpu.sync_copy(iv, idx_hbm.at[:, pl.ds(base, per_w)])
        pltpu.sync_copy(wv, w_hbm.at[:, pl.ds(base, per_w)])

    return topk_kernel


_STRIPE_SIZES = (20480, 8192, 4096)


def kernel(hidden_states, weight):
    bsz, seq_len, h = hidden_states.shape
    t = bsz * seq_len
    flat = hidden_states.reshape(t, h)
    topk_cache = {}
    idx_parts = []
    w_parts = []
    start = 0
    for size in _STRIPE_SIZES:
        if size not in topk_cache:
            topk_cache[size] = _make_topk_sc(size)
        logits_t = _router_logits_t(flat, weight, start, size)
        idx_t, w_t = topk_cache[size](logits_t)
        idx_parts.append(idx_t.T)
        w_parts.append(w_t.T)
        start += size
    topk_idx = jnp.concatenate(idx_parts, axis=0)
    topk_weight = jnp.concatenate(w_parts, axis=0)
    aux_loss = jnp.float32(0.0)
    return (topk_idx, topk_weight, aux_loss)


# speedup vs baseline: 1.0284x; 1.0015x over previous
import functools

import jax
import jax.numpy as jnp
from jax import lax
from jax.experimental import pallas as pl
from jax.experimental.pallas import tpu as pltpu
from jax.experimental.pallas import tpu_sc as plsc

HIDDEN = 4096
N_EXPERTS = 64
TOP_K = 8
LANES = 16
NUM_WORKERS = 32

_TC_BLOCK_T = 1024


def _logits_body(x_ref, w_ref, out_ref):
    out_ref[...] = lax.dot_general(
        w_ref[...], x_ref[...],
        dimension_numbers=(((1,), (1,)), ((), ())),
        preferred_element_type=jnp.float32)


def _router_logits_t(flat, weight, start, size):
    blk_off = start // _TC_BLOCK_T
    return pl.pallas_call(
        _logits_body,
        grid=(size // _TC_BLOCK_T,),
        in_specs=[
            pl.BlockSpec((_TC_BLOCK_T, HIDDEN), lambda i: (i + blk_off, 0)),
            pl.BlockSpec((N_EXPERTS, HIDDEN), lambda i: (0, 0)),
        ],
        out_specs=pl.BlockSpec((N_EXPERTS, _TC_BLOCK_T), lambda i: (0, i)),
        out_shape=jax.ShapeDtypeStruct((N_EXPERTS, size), jnp.float32),
    )(flat, weight)


def _make_topk_sc(t):
    per_w = t // NUM_WORKERS
    n_groups = per_w // LANES

    mesh = plsc.VectorSubcoreMesh(core_axis_name="c", subcore_axis_name="s")

    @functools.partial(
        pl.kernel,
        out_type=(
            jax.ShapeDtypeStruct((TOP_K, t), jnp.int32),
            jax.ShapeDtypeStruct((TOP_K, t), jnp.float32),
        ),
        mesh=mesh,
        scratch_types=[
            pltpu.VMEM((N_EXPERTS, per_w), jnp.float32),
            pltpu.VMEM((TOP_K, per_w), jnp.int32),
            pltpu.VMEM((TOP_K, per_w), jnp.float32),
        ],
    )
    def topk_kernel(logits_hbm, idx_hbm, w_hbm, lv, iv, wv):
        wid = lax.axis_index("s") * 2 + lax.axis_index("c")
        base = wid * per_w
        pltpu.sync_copy(logits_hbm.at[:, pl.ds(base, per_w)], lv)

        def group_body(g, carry):
            goff = g * LANES

            def chunk_body(c, state):
                tv = list(state[:TOP_K])
                ti = list(state[TOP_K:])
                e_hi = (7 - c) * 8
                for k in range(7, -1, -1):
                    e = e_hi + k
                    v = lv[e, pl.ds(goff, LANES)]
                    idx = jnp.broadcast_to(e, (LANES,)).astype(jnp.int32)
                    for j in range(TOP_K):
                        m = v >= tv[j]
                        hi = jnp.maximum(tv[j], v)
                        lo = jnp.minimum(tv[j], v)
                        ni = jnp.where(m, idx, ti[j])
                        idx = jnp.where(m, ti[j], idx)
                        tv[j] = hi
                        ti[j] = ni
                        v = lo
                return tuple(tv) + tuple(ti)

            init = tuple(jnp.full((LANES,), -jnp.inf, jnp.float32)
                         for _ in range(TOP_K)) + \
                tuple(jnp.zeros((LANES,), jnp.int32) for _ in range(TOP_K))
            state = lax.fori_loop(0, 8, chunk_body, init)
            tv = list(state[:TOP_K])
            ti = list(state[TOP_K:])
            mx = tv[0]
            es = [jnp.exp(tv[j] - mx) for j in range(TOP_K)]
            s = es[0]
            for j in range(1, TOP_K):
                s = s + es[j]
            r = 1.0 / s
            for j in range(TOP_K):
                iv[j, pl.ds(goff, LANES)] = ti[j]
                wv[j, pl.ds(goff, LANES)] = es[j] * r
            return carry

        lax.fori_loop(0, n_groups, group_body, 0)
        pltpu.sync_copy(iv, idx_hbm.at[:, pl.ds(base, per_w)])
        pltpu.sync_copy(wv, w_hbm.at[:, pl.ds(base, per_w)])

    return topk_kernel


_STRIPE_SIZES = (24576, 4096, 4096)


def kernel(hidden_states, weight):
    bsz, seq_len, h = hidden_states.shape
    t = bsz * seq_len
    flat = hidden_states.reshape(t, h)
    topk_cache = {}
    idx_parts = []
    w_parts = []
    start = 0
    for size in _STRIPE_SIZES:
        if size not in topk_cache:
            topk_cache[size] = _make_topk_sc(size)
        logits_t = _router_logits_t(flat, weight, start, size)
        idx_t, w_t = topk_cache[size](logits_t)
        idx_parts.append(idx_t.T)
        w_parts.append(w_t.T)
        start += size
    topk_idx = jnp.concatenate(idx_parts, axis=0)
    topk_weight = jnp.concatenate(w_parts, axis=0)
    aux_loss = jnp.float32(0.0)
    return (topk_idx, topk_weight, aux_loss)


# speedup vs baseline: 1.0289x; 1.0005x over previous
import functools

import jax
import jax.numpy as jnp
from jax import lax
from jax.experimental import pallas as pl
from jax.experimental.pallas import tpu as pltpu
from jax.experimental.pallas import tpu_sc as plsc

HIDDEN = 4096
N_EXPERTS = 64
TOP_K = 8
LANES = 16
NUM_WORKERS = 32

_TC_BLOCK_T = 1024


def _logits_body(x_ref, w_ref, out_ref):
    out_ref[...] = lax.dot_general(
        w_ref[...], x_ref[...],
        dimension_numbers=(((1,), (1,)), ((), ())),
        preferred_element_type=jnp.float32)


def _router_logits_t(flat, weight, start, size):
    blk_off = start // _TC_BLOCK_T
    return pl.pallas_call(
        _logits_body,
        grid=(size // _TC_BLOCK_T,),
        in_specs=[
            pl.BlockSpec((_TC_BLOCK_T, HIDDEN), lambda i: (i + blk_off, 0)),
            pl.BlockSpec((N_EXPERTS, HIDDEN), lambda i: (0, 0)),
        ],
        out_specs=pl.BlockSpec((N_EXPERTS, _TC_BLOCK_T), lambda i: (0, i)),
        out_shape=jax.ShapeDtypeStruct((N_EXPERTS, size), jnp.float32),
    )(flat, weight)


def _make_topk_sc(t):
    per_w = t // NUM_WORKERS
    n_groups = per_w // LANES

    mesh = plsc.VectorSubcoreMesh(core_axis_name="c", subcore_axis_name="s")

    @functools.partial(
        pl.kernel,
        out_type=(
            jax.ShapeDtypeStruct((TOP_K, t), jnp.int32),
            jax.ShapeDtypeStruct((TOP_K, t), jnp.float32),
        ),
        mesh=mesh,
        scratch_types=[
            pltpu.VMEM((N_EXPERTS, per_w), jnp.float32),
            pltpu.VMEM((TOP_K, per_w), jnp.int32),
            pltpu.VMEM((TOP_K, per_w), jnp.float32),
        ],
    )
    def topk_kernel(logits_hbm, idx_hbm, w_hbm, lv, iv, wv):
        wid = lax.axis_index("s") * 2 + lax.axis_index("c")
        base = wid * per_w
        pltpu.sync_copy(logits_hbm.at[:, pl.ds(base, per_w)], lv)

        def group_body(g, carry):
            goff = g * LANES

            def chunk_body(c, state):
                tv = list(state[:TOP_K])
                ti = list(state[TOP_K:])
                e_hi = (7 - c) * 8
                for k in range(7, -1, -1):
                    e = e_hi + k
                    v = lv[e, pl.ds(goff, LANES)]
                    idx = jnp.broadcast_to(e, (LANES,)).astype(jnp.int32)
                    for j in range(TOP_K):
                        m = v >= tv[j]
                        hi = jnp.maximum(tv[j], v)
                        lo = jnp.minimum(tv[j], v)
                        ni = jnp.where(m, idx, ti[j])
                        idx = jnp.where(m, ti[j], idx)
                        tv[j] = hi
                        ti[j] = ni
                        v = lo
                return tuple(tv) + tuple(ti)

            init = tuple(jnp.full((LANES,), -jnp.inf, jnp.float32)
                         for _ in range(TOP_K)) + \
                tuple(jnp.zeros((LANES,), jnp.int32) for _ in range(TOP_K))
            state = lax.fori_loop(0, 8, chunk_body, init)
            tv = list(state[:TOP_K])
            ti = list(state[TOP_K:])
            mx = tv[0]
            es = [jnp.exp(tv[j] - mx) for j in range(TOP_K)]
            s = es[0]
            for j in range(1, TOP_K):
                s = s + es[j]
            r = 1.0 / s
            for j in range(TOP_K):
                iv[j, pl.ds(goff, LANES)] = ti[j]
                wv[j, pl.ds(goff, LANES)] = es[j] * r
            return carry

        lax.fori_loop(0, n_groups, group_body, 0)
        pltpu.sync_copy(iv, idx_hbm.at[:, pl.ds(base, per_w)])
        pltpu.sync_copy(wv, w_hbm.at[:, pl.ds(base, per_w)])

    return topk_kernel


_STRIPE_SIZES = (20480, 8192, 4096)


def kernel(hidden_states, weight):
    bsz, seq_len, h = hidden_states.shape
    t = bsz * seq_len
    flat = hidden_states.reshape(t, h)
    topk_cache = {}
    idx_parts = []
    w_parts = []
    start = 0
    for size in _STRIPE_SIZES:
        if size not in topk_cache:
            topk_cache[size] = _make_topk_sc(size)
        logits_t = _router_logits_t(flat, weight, start, size)
        idx_t, w_t = topk_cache[size](logits_t)
        idx_parts.append(idx_t.T)
        w_parts.append(w_t.T)
        start += size
    topk_idx = jnp.concatenate(idx_parts, axis=0)
    topk_weight = jnp.concatenate(w_parts, axis=0)
    aux_loss = jnp.float32(0.0)
    return (topk_idx, topk_weight, aux_loss)
